# baseline (speedup 1.0000x reference)
import jax
import jax.numpy as jnp
from jax.experimental import pallas as pl
from jax.experimental.pallas import tpu as pltpu

BN = 2048
NSLOT = 3
S_IN = 2
S_OUT = 4


def _make_body(m, k, n):
    nfull = n // BN
    rem = n - nfull * BN
    wrows = BN // S_IN
    orows = m // S_OUT

    def body(x_ref, w_hbm, o_hbm, w_buf, o_buf, t_buf, in_sem, out_sem):
        def in_copies(j, slot):
            off = pl.multiple_of(j * BN, BN)
            return [
                pltpu.make_async_copy(
                    w_hbm.at[pl.ds(off + i * wrows, wrows), :],
                    w_buf.at[slot, pl.ds(i * wrows, wrows), :],
                    in_sem.at[slot, i],
                )
                for i in range(S_IN)
            ]

        def out_copies(j, slot):
            off = pl.multiple_of(j * BN, BN)
            return [
                pltpu.make_async_copy(
                    o_buf.at[slot, pl.ds(i * orows, orows), :],
                    o_hbm.at[pl.ds(i * orows, orows), pl.ds(off, BN)],
                    out_sem.at[slot, i],
                )
                for i in range(S_OUT)
            ]

        for j in range(min(NSLOT, nfull)):
            for c in in_copies(j, j % NSLOT):
                c.start()

        def step(j, carry):
            slot = jax.lax.rem(j, NSLOT)
            for c in in_copies(j, slot):
                c.wait()

            @pl.when(j >= NSLOT)
            def _():
                for c in out_copies(j - NSLOT, slot):
                    c.wait()

            o_buf[slot] = jax.lax.dot_general(
                x_ref[...],
                w_buf[slot].astype(jnp.bfloat16),
                dimension_numbers=(((1,), (1,)), ((), ())),
                preferred_element_type=jnp.float32,
            )
            for c in out_copies(j, slot):
                c.start()

            @pl.when(j + NSLOT < nfull)
            def _():
                for c in in_copies(j + NSLOT, slot):
                    c.start()

            return carry

        jax.lax.fori_loop(0, nfull, step, 0)

        if rem:
            tail_in = pltpu.make_async_copy(
                w_hbm.at[pl.ds(nfull * BN, rem), :],
                w_buf.at[0, pl.ds(0, rem), :],
                in_sem.at[0, 0],
            )
            tail_in.start()

        for j in range(max(nfull - NSLOT, 0), nfull):
            for c in out_copies(j, j % NSLOT):
                c.wait()

        if rem:
            tail_in.wait()
            t_buf[...] = jax.lax.dot_general(
                x_ref[...],
                w_buf[0, :rem, :].astype(jnp.bfloat16),
                dimension_numbers=(((1,), (1,)), ((), ())),
                preferred_element_type=jnp.float32,
            )
            tail_outs = [
                pltpu.make_async_copy(
                    t_buf.at[pl.ds(i * orows, orows), :],
                    o_hbm.at[pl.ds(i * orows, orows), pl.ds(nfull * BN, rem)],
                    out_sem.at[0, i],
                )
                for i in range(S_OUT)
            ]
            for c in tail_outs:
                c.start()
            for c in tail_outs:
                c.wait()

    return body


def kernel(total_features, norm_weight):
    m, k = total_features.shape
    n = norm_weight.shape[0]
    rem = n - (n // BN) * BN
    x16 = total_features.astype(jnp.bfloat16)
    return pl.pallas_call(
        _make_body(m, k, n),
        in_specs=[
            pl.BlockSpec(memory_space=pltpu.VMEM),
            pl.BlockSpec(memory_space=pl.MemorySpace.ANY),
        ],
        out_specs=pl.BlockSpec(memory_space=pl.MemorySpace.ANY),
        out_shape=jax.ShapeDtypeStruct((m, n), jnp.float32),
        scratch_shapes=[
            pltpu.VMEM((NSLOT, BN, k), jnp.float32),
            pltpu.VMEM((NSLOT, m, BN), jnp.float32),
            pltpu.VMEM((m, max(rem, 8)), jnp.float32),
            pltpu.SemaphoreType.DMA((NSLOT, S_IN)),
            pltpu.SemaphoreType.DMA((NSLOT, S_OUT)),
        ],
        compiler_params=pltpu.CompilerParams(
            vmem_limit_bytes=60 * 1024 * 1024,
        ),
    )(x16, norm_weight)


# speedup vs baseline: 1.0015x; 1.0015x over previous
import jax
import jax.numpy as jnp
from jax.experimental import pallas as pl
from jax.experimental.pallas import tpu as pltpu

BN = 2048
NSLOT = 3
W_SPLIT = 2
O_SPLIT = 4


def _make_body(m, k, n):
    nfull = n // BN
    rem = n - nfull * BN
    wrows = BN // W_SPLIT
    orows = m // O_SPLIT
    trows0 = min(rem, wrows)
    trows1 = rem - trows0

    def body(x_ref, w_hbm, o_hbm, w_a, w_b, o_0, o_1, o_2, o_3, t_buf,
             in_sem, out_sem):
        w_bufs = (w_a, w_b)
        o_bufs = (o_0, o_1, o_2, o_3)

        def in_copies(j, slot):
            off = pl.multiple_of(j * BN, BN)
            return [
                pltpu.make_async_copy(
                    w_hbm.at[pl.ds(off + h * wrows, wrows), :],
                    w_bufs[h].at[slot],
                    in_sem.at[slot, h],
                )
                for h in range(W_SPLIT)
            ]

        def out_copies(j, slot):
            off = pl.multiple_of(j * BN, BN)
            return [
                pltpu.make_async_copy(
                    o_bufs[q].at[slot],
                    o_hbm.at[pl.ds(q * orows, orows), pl.ds(off, BN)],
                    out_sem.at[slot, q],
                )
                for q in range(O_SPLIT)
            ]

        for j in range(min(NSLOT, nfull)):
            for c in in_copies(j, j % NSLOT):
                c.start()

        def step(j, carry):
            slot = jax.lax.rem(j, NSLOT)
            for c in in_copies(j, slot):
                c.wait()

            @pl.when(j >= NSLOT)
            def _():
                for c in out_copies(j - NSLOT, slot):
                    c.wait()

            for q in range(O_SPLIT):
                xq = x_ref[q * orows:(q + 1) * orows, :]
                for h in range(W_SPLIT):
                    o_bufs[q][slot, :, h * wrows:(h + 1) * wrows] = (
                        jax.lax.dot_general(
                            xq,
                            w_bufs[h][slot].astype(jnp.bfloat16),
                            dimension_numbers=(((1,), (1,)), ((), ())),
                            preferred_element_type=jnp.float32,
                        )
                    )
            for c in out_copies(j, slot):
                c.start()

            @pl.when(j + NSLOT < nfull)
            def _():
                for c in in_copies(j + NSLOT, slot):
                    c.start()

            return carry

        jax.lax.fori_loop(0, nfull, step, 0)

        if rem:
            tail_ins = [
                pltpu.make_async_copy(
                    w_hbm.at[pl.ds(nfull * BN, trows0), :],
                    w_a.at[0, pl.ds(0, trows0)],
                    in_sem.at[0, 0],
                )
            ]
            if trows1:
                tail_ins.append(
                    pltpu.make_async_copy(
                        w_hbm.at[pl.ds(nfull * BN + trows0, trows1), :],
                        w_b.at[0, pl.ds(0, trows1)],
                        in_sem.at[0, 1],
                    )
                )
            for c in tail_ins:
                c.start()

        for j in range(max(nfull - NSLOT, 0), nfull):
            for c in out_copies(j, j % NSLOT):
                c.wait()

        if rem:
            for c in tail_ins:
                c.wait()
            for q in range(O_SPLIT):
                xq = x_ref[q * orows:(q + 1) * orows, :]
                t_buf[q * orows:(q + 1) * orows, :trows0] = jax.lax.dot_general(
                    xq,
                    w_a[0, :trows0, :].astype(jnp.bfloat16),
                    dimension_numbers=(((1,), (1,)), ((), ())),
                    preferred_element_type=jnp.float32,
                )
                if trows1:
                    t_buf[q * orows:(q + 1) * orows, trows0:] = (
                        jax.lax.dot_general(
                            xq,
                            w_b[0, :trows1, :].astype(jnp.bfloat16),
                            dimension_numbers=(((1,), (1,)), ((), ())),
                            preferred_element_type=jnp.float32,
                        )
                    )
            tail_outs = [
                pltpu.make_async_copy(
                    t_buf.at[pl.ds(q * orows, orows), :],
                    o_hbm.at[pl.ds(q * orows, orows), pl.ds(nfull * BN, rem)],
                    out_sem.at[0, q],
                )
                for q in range(O_SPLIT)
            ]
            for c in tail_outs:
                c.start()
            for c in tail_outs:
                c.wait()

    return body


def kernel(total_features, norm_weight):
    m, k = total_features.shape
    n = norm_weight.shape[0]
    rem = n - (n // BN) * BN
    x16 = total_features.astype(jnp.bfloat16)
    return pl.pallas_call(
        _make_body(m, k, n),
        in_specs=[
            pl.BlockSpec(memory_space=pltpu.VMEM),
            pl.BlockSpec(memory_space=pl.MemorySpace.ANY),
        ],
        out_specs=pl.BlockSpec(memory_space=pl.MemorySpace.ANY),
        out_shape=jax.ShapeDtypeStruct((m, n), jnp.float32),
        scratch_shapes=[
            pltpu.VMEM((NSLOT, BN // W_SPLIT, k), jnp.float32),
            pltpu.VMEM((NSLOT, BN // W_SPLIT, k), jnp.float32),
            pltpu.VMEM((NSLOT, m // O_SPLIT, BN), jnp.float32),
            pltpu.VMEM((NSLOT, m // O_SPLIT, BN), jnp.float32),
            pltpu.VMEM((NSLOT, m // O_SPLIT, BN), jnp.float32),
            pltpu.VMEM((NSLOT, m // O_SPLIT, BN), jnp.float32),
            pltpu.VMEM((m, max(rem, 8)), jnp.float32),
            pltpu.SemaphoreType.DMA((NSLOT, W_SPLIT)),
            pltpu.SemaphoreType.DMA((NSLOT, O_SPLIT)),
        ],
        compiler_params=pltpu.CompilerParams(
            vmem_limit_bytes=60 * 1024 * 1024,
        ),
    )(x16, norm_weight)


# speedup vs baseline: 1.0031x; 1.0016x over previous
import jax
import jax.numpy as jnp
from jax.experimental import pallas as pl
from jax.experimental.pallas import tpu as pltpu

BN = 2048
NSLOT = 3
W_SPLIT = 2
O_SPLIT = 4


def _make_body(m, k, n):
    nfull = n // BN
    rem = n - nfull * BN
    wrows = BN // W_SPLIT
    orows = m // O_SPLIT
    trows0 = min(rem, wrows)
    trows1 = rem - trows0

    def body(x_ref, w_hbm, o_hbm, w_a, w_b, o_0, o_1, o_2, o_3, t_buf,
             in_sem, out_sem):
        w_bufs = (w_a, w_b)
        o_bufs = (o_0, o_1, o_2, o_3)

        def in_copies(j, slot):
            off = pl.multiple_of(j * BN, BN)
            return [
                pltpu.make_async_copy(
                    w_hbm.at[pl.ds(off + h * wrows, wrows), :],
                    w_bufs[h].at[slot],
                    in_sem.at[slot, h],
                )
                for h in range(W_SPLIT)
            ]

        def out_copies(j, slot):
            off = pl.multiple_of(j * BN, BN)
            return [
                pltpu.make_async_copy(
                    o_bufs[q].at[slot],
                    o_hbm.at[pl.ds(q * orows, orows), pl.ds(off, BN)],
                    out_sem.at[slot, q],
                )
                for q in range(O_SPLIT)
            ]

        for j in range(min(NSLOT, nfull)):
            for h, c in enumerate(in_copies(j, j % NSLOT)):
                c.start(priority=h % 2)

        def step(j, carry):
            slot = jax.lax.rem(j, NSLOT)
            for c in in_copies(j, slot):
                c.wait()

            @pl.when(j >= NSLOT)
            def _():
                for c in out_copies(j - NSLOT, slot):
                    c.wait()

            for q in range(O_SPLIT):
                xq = x_ref[q * orows:(q + 1) * orows, :]
                for h in range(W_SPLIT):
                    o_bufs[q][slot, :, h * wrows:(h + 1) * wrows] = (
                        jax.lax.dot_general(
                            xq,
                            w_bufs[h][slot].astype(jnp.bfloat16),
                            dimension_numbers=(((1,), (1,)), ((), ())),
                            preferred_element_type=jnp.float32,
                        )
                    )
            for q, c in enumerate(out_copies(j, slot)):
                c.start(priority=q % 2)

            @pl.when(j + NSLOT < nfull)
            def _():
                for h, c in enumerate(in_copies(j + NSLOT, slot)):
                    c.start(priority=h % 2)

            return carry

        jax.lax.fori_loop(0, nfull, step, 0)

        if rem:
            tail_ins = [
                pltpu.make_async_copy(
                    w_hbm.at[pl.ds(nfull * BN, trows0), :],
                    w_a.at[0, pl.ds(0, trows0)],
                    in_sem.at[0, 0],
                )
            ]
            if trows1:
                tail_ins.append(
                    pltpu.make_async_copy(
                        w_hbm.at[pl.ds(nfull * BN + trows0, trows1), :],
                        w_b.at[0, pl.ds(0, trows1)],
                        in_sem.at[0, 1],
                    )
                )
            for h, c in enumerate(tail_ins):
                c.start(priority=h % 2)

        for j in range(max(nfull - NSLOT, 0), nfull):
            for c in out_copies(j, j % NSLOT):
                c.wait()

        if rem:
            for c in tail_ins:
                c.wait()
            for q in range(O_SPLIT):
                xq = x_ref[q * orows:(q + 1) * orows, :]
                t_buf[q * orows:(q + 1) * orows, :trows0] = jax.lax.dot_general(
                    xq,
                    w_a[0, :trows0, :].astype(jnp.bfloat16),
                    dimension_numbers=(((1,), (1,)), ((), ())),
                    preferred_element_type=jnp.float32,
                )
                if trows1:
                    t_buf[q * orows:(q + 1) * orows, trows0:] = (
                        jax.lax.dot_general(
                            xq,
                            w_b[0, :trows1, :].astype(jnp.bfloat16),
                            dimension_numbers=(((1,), (1,)), ((), ())),
                            preferred_element_type=jnp.float32,
                        )
                    )
            tail_outs = [
                pltpu.make_async_copy(
                    t_buf.at[pl.ds(q * orows, orows), :],
                    o_hbm.at[pl.ds(q * orows, orows), pl.ds(nfull * BN, rem)],
                    out_sem.at[0, q],
                )
                for q in range(O_SPLIT)
            ]
            for q, c in enumerate(tail_outs):
                c.start(priority=q % 2)
            for c in tail_outs:
                c.wait()

    return body


def kernel(total_features, norm_weight):
    m, k = total_features.shape
    n = norm_weight.shape[0]
    rem = n - (n // BN) * BN
    x16 = total_features.astype(jnp.bfloat16)
    return pl.pallas_call(
        _make_body(m, k, n),
        in_specs=[
            pl.BlockSpec(memory_space=pltpu.VMEM),
            pl.BlockSpec(memory_space=pl.MemorySpace.ANY),
        ],
        out_specs=pl.BlockSpec(memory_space=pl.MemorySpace.ANY),
        out_shape=jax.ShapeDtypeStruct((m, n), jnp.float32),
        scratch_shapes=[
            pltpu.VMEM((NSLOT, BN // W_SPLIT, k), jnp.float32),
            pltpu.VMEM((NSLOT, BN // W_SPLIT, k), jnp.float32),
            pltpu.VMEM((NSLOT, m // O_SPLIT, BN), jnp.float32),
            pltpu.VMEM((NSLOT, m // O_SPLIT, BN), jnp.float32),
            pltpu.VMEM((NSLOT, m // O_SPLIT, BN), jnp.float32),
            pltpu.VMEM((NSLOT, m // O_SPLIT, BN), jnp.float32),
            pltpu.VMEM((m, max(rem, 8)), jnp.float32),
            pltpu.SemaphoreType.DMA((NSLOT, W_SPLIT)),
            pltpu.SemaphoreType.DMA((NSLOT, O_SPLIT)),
        ],
        compiler_params=pltpu.CompilerParams(
            vmem_limit_bytes=60 * 1024 * 1024,
        ),
    )(x16, norm_weight)


# speedup vs baseline: 1.0040x; 1.0009x over previous
import jax
import jax.numpy as jnp
from jax.experimental import pallas as pl
from jax.experimental.pallas import tpu as pltpu

BN = 2048
NSLOT = 3
W_SPLIT = 2
O_SPLIT = 4
NCORES = 2


def _make_body(m, k, n):
    nfull = n // BN
    rem = n - nfull * BN
    per_core = nfull // NCORES
    wrows = BN // W_SPLIT
    orows = m // O_SPLIT
    trows0 = min(rem, wrows)
    trows1 = rem - trows0

    def body(x_hbm, w_hbm, o_hbm, x_buf, w_a, w_b, o_0, o_1, o_2, o_3,
             t_buf, x_sem, in_sem, out_sem):
        core = jax.lax.axis_index("core")
        base = core * per_core
        w_bufs = (w_a, w_b)
        o_bufs = (o_0, o_1, o_2, o_3)

        x_copy = pltpu.make_async_copy(x_hbm, x_buf, x_sem)
        x_copy.start()

        def in_copies(jb, slot):
            off = pl.multiple_of(jb * BN, BN)
            return [
                pltpu.make_async_copy(
                    w_hbm.at[pl.ds(off + h * wrows, wrows), :],
                    w_bufs[h].at[slot],
                    in_sem.at[slot, h],
                )
                for h in range(W_SPLIT)
            ]

        def out_copies(jb, slot):
            off = pl.multiple_of(jb * BN, BN)
            return [
                pltpu.make_async_copy(
                    o_bufs[q].at[slot],
                    o_hbm.at[pl.ds(q * orows, orows), pl.ds(off, BN)],
                    out_sem.at[slot, q],
                )
                for q in range(O_SPLIT)
            ]

        for j in range(min(NSLOT, per_core)):
            for h, c in enumerate(in_copies(base + j, j % NSLOT)):
                c.start(priority=h % 2)

        x_copy.wait()

        def step(j, carry):
            slot = jax.lax.rem(j, NSLOT)
            jb = base + j
            for c in in_copies(jb, slot):
                c.wait()

            @pl.when(j >= NSLOT)
            def _():
                for c in out_copies(jb - NSLOT, slot):
                    c.wait()

            for q in range(O_SPLIT):
                xq = x_buf[q * orows:(q + 1) * orows, :]
                for h in range(W_SPLIT):
                    o_bufs[q][slot, :, h * wrows:(h + 1) * wrows] = (
                        jax.lax.dot_general(
                            xq,
                            w_bufs[h][slot].astype(jnp.bfloat16),
                            dimension_numbers=(((1,), (1,)), ((), ())),
                            preferred_element_type=jnp.float32,
                        )
                    )
            for q, c in enumerate(out_copies(jb, slot)):
                c.start(priority=q % 2)

            @pl.when(j + NSLOT < per_core)
            def _():
                for h, c in enumerate(in_copies(jb + NSLOT, slot)):
                    c.start(priority=h % 2)

            return carry

        jax.lax.fori_loop(0, per_core, step, 0)

        if rem:
            @pl.when(core == NCORES - 1)
            def _():
                tail_in0 = pltpu.make_async_copy(
                    w_hbm.at[pl.ds(nfull * BN, trows0), :],
                    w_a.at[0, pl.ds(0, trows0)],
                    in_sem.at[0, 0],
                )
                tail_in0.start()
                if trows1:
                    tail_in1 = pltpu.make_async_copy(
                        w_hbm.at[pl.ds(nfull * BN + trows0, trows1), :],
                        w_b.at[0, pl.ds(0, trows1)],
                        in_sem.at[0, 1],
                    )
                    tail_in1.start(priority=1)

                for j in range(max(per_core - NSLOT, 0), per_core):
                    for c in out_copies(base + j, j % NSLOT):
                        c.wait()

                tail_in0.wait()
                if trows1:
                    tail_in1.wait()
                for q in range(O_SPLIT):
                    xq = x_buf[q * orows:(q + 1) * orows, :]
                    t_buf[q * orows:(q + 1) * orows, :trows0] = (
                        jax.lax.dot_general(
                            xq,
                            w_a[0, :trows0, :].astype(jnp.bfloat16),
                            dimension_numbers=(((1,), (1,)), ((), ())),
                            preferred_element_type=jnp.float32,
                        )
                    )
                    if trows1:
                        t_buf[q * orows:(q + 1) * orows, trows0:] = (
                            jax.lax.dot_general(
                                xq,
                                w_b[0, :trows1, :].astype(jnp.bfloat16),
                                dimension_numbers=(((1,), (1,)), ((), ())),
                                preferred_element_type=jnp.float32,
                            )
                        )
                tail_outs = [
                    pltpu.make_async_copy(
                        t_buf.at[pl.ds(q * orows, orows), :],
                        o_hbm.at[pl.ds(q * orows, orows),
                                 pl.ds(nfull * BN, rem)],
                        out_sem.at[0, q],
                    )
                    for q in range(O_SPLIT)
                ]
                for q, c in enumerate(tail_outs):
                    c.start(priority=q % 2)
                for c in tail_outs:
                    c.wait()

            @pl.when(core != NCORES - 1)
            def _():
                for j in range(max(per_core - NSLOT, 0), per_core):
                    for c in out_copies(base + j, j % NSLOT):
                        c.wait()
        else:
            for j in range(max(per_core - NSLOT, 0), per_core):
                for c in out_copies(base + j, j % NSLOT):
                    c.wait()

    return body


def kernel(total_features, norm_weight):
    m, k = total_features.shape
    n = norm_weight.shape[0]
    rem = n - (n // BN) * BN
    x16 = total_features.astype(jnp.bfloat16)
    mesh = pltpu.create_tensorcore_mesh("core", num_cores=NCORES)
    run = pl.kernel(
        _make_body(m, k, n),
        out_type=jax.ShapeDtypeStruct((m, n), jnp.float32),
        mesh=mesh,
        scratch_types=[
            pltpu.VMEM((m, k), jnp.bfloat16),
            pltpu.VMEM((NSLOT, BN // W_SPLIT, k), jnp.float32),
            pltpu.VMEM((NSLOT, BN // W_SPLIT, k), jnp.float32),
            pltpu.VMEM((NSLOT, m // O_SPLIT, BN), jnp.float32),
            pltpu.VMEM((NSLOT, m // O_SPLIT, BN), jnp.float32),
            pltpu.VMEM((NSLOT, m // O_SPLIT, BN), jnp.float32),
            pltpu.VMEM((NSLOT, m // O_SPLIT, BN), jnp.float32),
            pltpu.VMEM((m, max(rem, 8)), jnp.float32),
            pltpu.SemaphoreType.DMA,
            pltpu.SemaphoreType.DMA((NSLOT, W_SPLIT)),
            pltpu.SemaphoreType.DMA((NSLOT, O_SPLIT)),
        ],
        compiler_params=pltpu.CompilerParams(
            vmem_limit_bytes=60 * 1024 * 1024,
        ),
    )
    return run(x16, norm_weight)


# speedup vs baseline: 1.0061x; 1.0021x over previous
import jax
import jax.numpy as jnp
from jax.experimental import pallas as pl
from jax.experimental.pallas import tpu as pltpu

BN = 4096


def _mm_kernel(x_ref, w_ref, o_ref):
    o_ref[...] = jax.lax.dot_general(
        x_ref[...],
        w_ref[...].astype(jnp.bfloat16),
        dimension_numbers=(((1,), (1,)), ((), ())),
        preferred_element_type=jnp.float32,
    )


def kernel(total_features, norm_weight):
    m, k = total_features.shape
    n = norm_weight.shape[0]
    x16 = total_features.astype(jnp.bfloat16)
    return pl.pallas_call(
        _mm_kernel,
        grid=(pl.cdiv(n, BN),),
        in_specs=[
            pl.BlockSpec((m, k), lambda j: (0, 0)),
            pl.BlockSpec((BN, k), lambda j: (j, 0)),
        ],
        out_specs=pl.BlockSpec((m, BN), lambda j: (0, j)),
        out_shape=jax.ShapeDtypeStruct((m, n), jnp.float32),
        compiler_params=pltpu.CompilerParams(
            dimension_semantics=("arbitrary",),
        ),
    )(x16, norm_weight)
